# Initial kernel scaffold; baseline (speedup 1.0000x reference)
import jax
import jax.numpy as jnp
from jax.experimental import pallas as pl


def kernel(x, edge_index, edge_type, W0, Wself0, b0, W1, Wself1, b1):
    raise NotImplementedError("write your pallas kernel here")


# speedup vs baseline: 1.3548x; 1.3548x over previous
import functools

import jax
import jax.numpy as jnp
from jax import lax
from jax.experimental import pallas as pl
from jax.experimental.pallas import tpu as pltpu
from jax.experimental.pallas import tpu_sc as plsc

_K = 128
_NTILES = 32


@functools.cache
def _build(N, D, R, E):
    EPT = -(-E // (_NTILES * _K)) * _K
    NCHUNK = EPT // _K
    EPAD = EPT * _NTILES
    NP = -(-(N + 1) // 128) * 128
    ZPT = NP // 16
    ZROWS = 64
    BN = 1000

    def tr_body(h_ref, w_ref, o_ref):
        o_ref[0] = jnp.dot(h_ref[...], w_ref[0],
                           preferred_element_type=jnp.float32)

    tc_transform = pl.pallas_call(
        tr_body,
        grid=(N // BN, R),
        in_specs=[
            pl.BlockSpec((BN, D), lambda i, j: (i, 0)),
            pl.BlockSpec((1, D, D), lambda i, j: (j, 0, 0)),
        ],
        out_specs=pl.BlockSpec((1, BN, D), lambda i, j: (j, i, 0)),
        out_shape=jax.ShapeDtypeStruct((R, N, D), jnp.float32),
    )

    mesh = plsc.VectorSubcoreMesh(core_axis_name="c", subcore_axis_name="s")

    @functools.partial(
        pl.kernel,
        out_type=jax.ShapeDtypeStruct((2, NP, D), jnp.float32),
        mesh=mesh,
        scratch_types=[
            pltpu.VMEM((EPT,), jnp.int32),
            pltpu.VMEM((EPT,), jnp.int32),
            pltpu.VMEM((NCHUNK, _K), jnp.int32),
            pltpu.VMEM((NCHUNK, _K), jnp.int32),
            pltpu.VMEM((_K, D), jnp.float32),
            pltpu.VMEM((ZROWS, D), jnp.float32),
            pltpu.VMEM_SHARED((NP, D), jnp.float32),
            pltpu.SemaphoreType.DMA,
        ],
    )
    def sc_msgpass(hall, srcp, etp, dstp2, out, src_v, et_v, dst_v, gidx_v,
                   rows_v, zbuf, acc, sem):
        c = lax.axis_index("c")
        s = lax.axis_index("s")
        wid = s * 2 + c
        ebase = wid * EPT

        def zfill(i, carry):
            for j in range(D // 16):
                zbuf[i, pl.ds(j * 16, 16)] = jnp.zeros((16,), jnp.float32)
            return carry
        lax.fori_loop(0, ZROWS, zfill, 0)
        zrow = s * ZPT
        nfull = ZPT // ZROWS
        def zcopy(k, carry):
            pltpu.sync_copy(zbuf, acc.at[pl.ds(zrow + k * ZROWS, ZROWS)])
            return carry
        lax.fori_loop(0, nfull, zcopy, 0)
        rem = ZPT - nfull * ZROWS
        if rem:
            pltpu.sync_copy(zbuf.at[pl.ds(0, rem)],
                            acc.at[pl.ds(zrow + nfull * ZROWS, rem)])
        plsc.subcore_barrier()

        pltpu.sync_copy(srcp.at[pl.ds(ebase, EPT)], src_v)
        pltpu.sync_copy(etp.at[pl.ds(ebase, EPT)], et_v)
        pltpu.sync_copy(dstp2.at[pl.ds(wid * NCHUNK, NCHUNK)], dst_v)

        def gcompute(i, carry):
            for j in range(_K // 16):
                off = i * _K + j * 16
                g = et_v[pl.ds(off, 16)] * N + src_v[pl.ds(off, 16)]
                gidx_v[i, pl.ds(j * 16, 16)] = g
            return carry
        lax.fori_loop(0, NCHUNK, gcompute, 0)

        def echunk(i, carry):
            pltpu.async_copy(hall.at[gidx_v.at[i]], rows_v, sem).wait()
            pltpu.sync_copy(rows_v, acc.at[dst_v.at[i]], add=True)
            return carry
        lax.fori_loop(0, NCHUNK, echunk, 0)

        plsc.subcore_barrier()
        pltpu.sync_copy(acc.at[pl.ds(zrow, ZPT)], out.at[c, pl.ds(zrow, ZPT)])

    def mk_final(relu):
        def fin_body(p0_ref, p1_ref, h_ref, w_ref, b_ref, o_ref):
            v = (p0_ref[0] + p1_ref[0]
                 + jnp.dot(h_ref[...], w_ref[...],
                           preferred_element_type=jnp.float32)
                 + b_ref[...])
            if relu:
                v = jnp.maximum(v, 0.0)
            o_ref[...] = v

        return pl.pallas_call(
            fin_body,
            grid=(N // BN,),
            in_specs=[
                pl.BlockSpec((1, BN, D), lambda i: (0, i, 0)),
                pl.BlockSpec((1, BN, D), lambda i: (1, i, 0)),
                pl.BlockSpec((BN, D), lambda i: (i, 0)),
                pl.BlockSpec((D, D), lambda i: (0, 0)),
                pl.BlockSpec((1, D), lambda i: (0, 0)),
            ],
            out_specs=pl.BlockSpec((BN, D), lambda i: (i, 0)),
            out_shape=jax.ShapeDtypeStruct((N, D), jnp.float32),
        )

    tc_final_relu = mk_final(True)
    tc_final_lin = mk_final(False)

    def run(x, edge_index, edge_type, W0, Wself0, b0, W1, Wself1, b1):
        pad = EPAD - E
        src = edge_index[0]
        dst = edge_index[1]
        srcp = jnp.concatenate([src, jnp.zeros((pad,), jnp.int32)])
        etp = jnp.concatenate([edge_type, jnp.zeros((pad,), jnp.int32)])
        dstp2 = jnp.concatenate(
            [dst, jnp.full((pad,), N, jnp.int32)]).reshape(-1, _K)
        b0r = b0.reshape(1, D)
        b1r = b1.reshape(1, D)

        hall0 = tc_transform(x, W0).reshape(R * N, D)
        part0 = sc_msgpass(hall0, srcp, etp, dstp2)
        h = tc_final_relu(part0, part0, x, Wself0, b0r)
        hall1 = tc_transform(h, W1).reshape(R * N, D)
        part1 = sc_msgpass(hall1, srcp, etp, dstp2)
        return tc_final_lin(part1, part1, h, Wself1, b1r)

    return run


def kernel(x, edge_index, edge_type, W0, Wself0, b0, W1, Wself1, b1):
    N, D = x.shape
    R = W0.shape[0]
    E = edge_index.shape[1]
    return _build(N, D, R, E)(x, edge_index, edge_type, W0, Wself0, b0,
                              W1, Wself1, b1)


# speedup vs baseline: 1.3936x; 1.0286x over previous
import functools

import jax
import jax.numpy as jnp
from jax import lax
from jax.experimental import pallas as pl
from jax.experimental.pallas import tpu as pltpu
from jax.experimental.pallas import tpu_sc as plsc

_K = 128
_F0 = 0.72


@functools.cache
def _build(N, D, R, E):
    CT = -(-E // _K)
    n0 = max(2, int(round(CT * _F0 / 32)) * 2)
    n1 = max(2, 2 * (-(-(CT - 16 * n0) // 32)))
    CH0 = 16 * n0
    LCH = CH0 + 16 * n1
    LCHA = max(LCH, CH0 + 15 * n1 + n0)
    LPAD = LCHA * _K
    NP = -(-(N + 1) // 128) * 128
    ZPT = NP // 16
    BN = 1000

    def tr_body(h_ref, w_ref, o_ref):
        o_ref[0] = jnp.dot(h_ref[...], w_ref[0],
                           preferred_element_type=jnp.float32)

    tc_transform = pl.pallas_call(
        tr_body,
        grid=(N // BN, R),
        in_specs=[
            pl.BlockSpec((BN, D), lambda i, j: (i, 0)),
            pl.BlockSpec((1, D, D), lambda i, j: (j, 0, 0)),
        ],
        out_specs=pl.BlockSpec((1, BN, D), lambda i, j: (j, i, 0)),
        out_shape=jax.ShapeDtypeStruct((R, N, D), jnp.float32),
    )

    mesh = plsc.VectorSubcoreMesh(core_axis_name="c", subcore_axis_name="s")

    @functools.partial(
        pl.kernel,
        out_type=jax.ShapeDtypeStruct((2, NP, D), jnp.float32),
        mesh=mesh,
        scratch_types=[
            pltpu.VMEM((n0, _K), jnp.int32),
            pltpu.VMEM((n0, _K), jnp.int32),
            pltpu.VMEM((_K, D), jnp.float32),
            pltpu.VMEM((_K, D), jnp.float32),
            pltpu.VMEM_SHARED((NP, D), jnp.float32),
            pltpu.SemaphoreType.DMA,
            pltpu.SemaphoreType.DMA,
            pltpu.SemaphoreType.DMA,
        ],
    )
    def sc_msgpass(hall, gix2, dstp2, out, dst_v, gidx_v, rows_a, rows_b,
                   acc, semz, sema, semb):
        c = lax.axis_index("c")
        s = lax.axis_index("s")
        zrow = s * ZPT
        n = jnp.where(c == 0, n0, n1)
        cbase = jnp.where(c == 0, s * n0, CH0 + s * n1)

        cp1 = pltpu.async_copy(gix2.at[pl.ds(cbase, n0)], gidx_v, semz)
        cp2 = pltpu.async_copy(dstp2.at[pl.ds(cbase, n0)], dst_v, semz)

        def zfill(i, carry):
            for j in range(D // 16):
                rows_a[i, pl.ds(j * 16, 16)] = jnp.zeros((16,), jnp.float32)
            return carry
        lax.fori_loop(0, _K, zfill, 0)
        nfull = ZPT // _K
        def zcopy(k, carry):
            pltpu.sync_copy(rows_a, acc.at[pl.ds(zrow + k * _K, _K)])
            return carry
        lax.fori_loop(0, nfull, zcopy, 0)
        rem = ZPT - nfull * _K
        if rem:
            pltpu.sync_copy(rows_a.at[pl.ds(0, rem)],
                            acc.at[pl.ds(zrow + nfull * _K, rem)])
        cp1.wait()
        cp2.wait()
        plsc.subcore_barrier()

        pltpu.async_copy(hall.at[gidx_v.at[0]], rows_a, sema)
        pltpu.async_copy(hall.at[gidx_v.at[1]], rows_b, semb)

        def pair(k, carry):
            pltpu.make_async_copy(hall.at[gidx_v.at[2 * k]],
                                  rows_a, sema).wait()
            pltpu.sync_copy(rows_a, acc.at[dst_v.at[2 * k]], add=True)

            @pl.when(k + 1 < n // 2)
            def _():
                pltpu.async_copy(hall.at[gidx_v.at[2 * k + 2]],
                                 rows_a, sema)

            pltpu.make_async_copy(hall.at[gidx_v.at[2 * k + 1]],
                                  rows_b, semb).wait()
            pltpu.sync_copy(rows_b, acc.at[dst_v.at[2 * k + 1]], add=True)

            @pl.when(k + 1 < n // 2)
            def _():
                pltpu.async_copy(hall.at[gidx_v.at[2 * k + 3]],
                                 rows_b, semb)
            return carry
        lax.fori_loop(0, n // 2, pair, 0)

        plsc.subcore_barrier()
        pltpu.sync_copy(acc.at[pl.ds(zrow, ZPT)],
                        out.at[c, pl.ds(zrow, ZPT)])

    def mk_final(relu):
        def fin_body(p0_ref, p1_ref, h_ref, w_ref, b_ref, o_ref):
            v = (p0_ref[0] + p1_ref[0]
                 + jnp.dot(h_ref[...], w_ref[...],
                           preferred_element_type=jnp.float32)
                 + b_ref[...])
            if relu:
                v = jnp.maximum(v, 0.0)
            o_ref[...] = v

        return pl.pallas_call(
            fin_body,
            grid=(N // BN,),
            in_specs=[
                pl.BlockSpec((1, BN, D), lambda i: (0, i, 0)),
                pl.BlockSpec((1, BN, D), lambda i: (1, i, 0)),
                pl.BlockSpec((BN, D), lambda i: (i, 0)),
                pl.BlockSpec((D, D), lambda i: (0, 0)),
                pl.BlockSpec((1, D), lambda i: (0, 0)),
            ],
            out_specs=pl.BlockSpec((BN, D), lambda i: (i, 0)),
            out_shape=jax.ShapeDtypeStruct((N, D), jnp.float32),
        )

    tc_final_relu = mk_final(True)
    tc_final_lin = mk_final(False)

    def run(x, edge_index, edge_type, W0, Wself0, b0, W1, Wself1, b1):
        pad = LPAD - E
        src = edge_index[0]
        dst = edge_index[1]
        gix = edge_type * N + src
        gix2 = jnp.concatenate(
            [gix, jnp.zeros((pad,), jnp.int32)]).reshape(LCHA, _K)
        dstp2 = jnp.concatenate(
            [dst, jnp.full((pad,), N, jnp.int32)]).reshape(LCHA, _K)
        b0r = b0.reshape(1, D)
        b1r = b1.reshape(1, D)

        hall0 = tc_transform(x, W0).reshape(R * N, D)
        part0 = sc_msgpass(hall0, gix2, dstp2)
        h = tc_final_relu(part0, part0, x, Wself0, b0r)
        hall1 = tc_transform(h, W1).reshape(R * N, D)
        part1 = sc_msgpass(hall1, gix2, dstp2)
        return tc_final_lin(part1, part1, h, Wself1, b1r)

    return run


def kernel(x, edge_index, edge_type, W0, Wself0, b0, W1, Wself1, b1):
    N, D = x.shape
    R = W0.shape[0]
    E = edge_index.shape[1]
    return _build(N, D, R, E)(x, edge_index, edge_type, W0, Wself0, b0,
                              W1, Wself1, b1)


# speedup vs baseline: 1.5891x; 1.1403x over previous
import functools

import jax
import jax.numpy as jnp
from jax import lax
from jax.experimental import pallas as pl
from jax.experimental.pallas import tpu as pltpu
from jax.experimental.pallas import tpu_sc as plsc

_K = 128
_F0 = 0.845


@functools.cache
def _build(N, D, R, E):
    CT = -(-E // _K)
    n0 = max(8, int(round(CT * _F0 / 128)) * 8)
    n0 = min(n0, 64)
    n1 = max(8, 8 * (-(-(CT - 16 * n0) // 128)))
    CH0 = 16 * n0
    LCH = CH0 + 16 * n1
    LCHA = max(LCH, CH0 + 15 * n1 + n0)
    LPAD = LCHA * _K
    NP = -(-(N + 1) // 128) * 128
    ZPT = NP // 16
    BN = 1000

    def tr_body(h_ref, w_ref, o_ref):
        o_ref[0] = jnp.dot(h_ref[...], w_ref[0],
                           preferred_element_type=jnp.float32)

    tc_transform = pl.pallas_call(
        tr_body,
        grid=(N // BN, R),
        in_specs=[
            pl.BlockSpec((BN, D), lambda i, j: (i, 0)),
            pl.BlockSpec((1, D, D), lambda i, j: (j, 0, 0)),
        ],
        out_specs=pl.BlockSpec((1, BN, D), lambda i, j: (j, i, 0)),
        out_shape=jax.ShapeDtypeStruct((R, N, D), jnp.float32),
    )

    mesh = plsc.VectorSubcoreMesh(core_axis_name="c", subcore_axis_name="s")

    @functools.partial(
        pl.kernel,
        out_type=jax.ShapeDtypeStruct((2, NP, D), jnp.float32),
        mesh=mesh,
        scratch_types=[
            pltpu.VMEM((n0, _K), jnp.int32),
            pltpu.VMEM((n0, _K), jnp.int32),
            pltpu.VMEM((_K, D), jnp.float32),
            pltpu.VMEM((_K, D), jnp.float32),
            pltpu.VMEM_SHARED((NP, D), jnp.float32),
            pltpu.SemaphoreType.DMA,
            pltpu.SemaphoreType.DMA,
            pltpu.SemaphoreType.DMA,
        ],
    )
    def sc_msgpass(hall, gix2, dstp2, out, dst_v, gidx_v, rows_a, rows_b,
                   acc, semz, sema, semb):
        c = lax.axis_index("c")
        s = lax.axis_index("s")
        zrow = s * ZPT
        n = jnp.where(c == 0, n0, n1)
        cbase = jnp.where(c == 0, s * n0, CH0 + s * n1)

        cp1 = pltpu.async_copy(gix2.at[pl.ds(cbase, n0)], gidx_v, semz)
        cp2 = pltpu.async_copy(dstp2.at[pl.ds(cbase, n0)], dst_v, semz)

        def zfill(i, carry):
            for j in range(D // 16):
                rows_a[i, pl.ds(j * 16, 16)] = jnp.zeros((16,), jnp.float32)
            return carry
        lax.fori_loop(0, _K, zfill, 0)
        nfull = ZPT // _K
        def zcopy(k, carry):
            pltpu.sync_copy(rows_a, acc.at[pl.ds(zrow + k * _K, _K)])
            return carry
        lax.fori_loop(0, nfull, zcopy, 0)
        rem = ZPT - nfull * _K
        if rem:
            pltpu.sync_copy(rows_a.at[pl.ds(0, rem)],
                            acc.at[pl.ds(zrow + nfull * _K, rem)])
        cp1.wait()
        cp2.wait()
        plsc.subcore_barrier()

        pltpu.async_copy(hall.at[gidx_v.at[0]], rows_a, sema)
        pltpu.async_copy(hall.at[gidx_v.at[1]], rows_b, semb)

        def pair(k, carry):
            pltpu.make_async_copy(hall.at[gidx_v.at[2 * k]],
                                  rows_a, sema).wait()
            pltpu.sync_copy(rows_a, acc.at[dst_v.at[2 * k]], add=True)

            @pl.when(k + 1 < n // 2)
            def _():
                pltpu.async_copy(hall.at[gidx_v.at[2 * k + 2]],
                                 rows_a, sema)

            pltpu.make_async_copy(hall.at[gidx_v.at[2 * k + 1]],
                                  rows_b, semb).wait()
            pltpu.sync_copy(rows_b, acc.at[dst_v.at[2 * k + 1]], add=True)

            @pl.when(k + 1 < n // 2)
            def _():
                pltpu.async_copy(hall.at[gidx_v.at[2 * k + 3]],
                                 rows_b, semb)
            return carry
        lax.fori_loop(0, n // 2, pair, 0)

        plsc.subcore_barrier()
        pltpu.sync_copy(acc.at[pl.ds(zrow, ZPT)],
                        out.at[c, pl.ds(zrow, ZPT)])

    def mk_final(relu):
        def fin_body(p0_ref, p1_ref, h_ref, w_ref, b_ref, o_ref):
            v = (p0_ref[0] + p1_ref[0]
                 + jnp.dot(h_ref[...], w_ref[...],
                           preferred_element_type=jnp.float32)
                 + b_ref[...])
            if relu:
                v = jnp.maximum(v, 0.0)
            o_ref[...] = v

        return pl.pallas_call(
            fin_body,
            grid=(N // BN,),
            in_specs=[
                pl.BlockSpec((1, BN, D), lambda i: (0, i, 0)),
                pl.BlockSpec((1, BN, D), lambda i: (1, i, 0)),
                pl.BlockSpec((BN, D), lambda i: (i, 0)),
                pl.BlockSpec((D, D), lambda i: (0, 0)),
                pl.BlockSpec((1, D), lambda i: (0, 0)),
            ],
            out_specs=pl.BlockSpec((BN, D), lambda i: (i, 0)),
            out_shape=jax.ShapeDtypeStruct((N, D), jnp.float32),
        )

    tc_final_relu = mk_final(True)
    tc_final_lin = mk_final(False)

    def run(x, edge_index, edge_type, W0, Wself0, b0, W1, Wself1, b1):
        pad = LPAD - E
        src = edge_index[0]
        dst = edge_index[1]
        gix = edge_type * N + src
        gix2 = jnp.concatenate(
            [gix, jnp.zeros((pad,), jnp.int32)]).reshape(LCHA, _K)
        dstp2 = jnp.concatenate(
            [dst, jnp.full((pad,), N, jnp.int32)]).reshape(LCHA, _K)
        b0r = b0.reshape(1, D)
        b1r = b1.reshape(1, D)

        hall0 = tc_transform(x, W0).reshape(R * N, D)
        part0 = sc_msgpass(hall0, gix2, dstp2)
        h = tc_final_relu(part0, part0, x, Wself0, b0r)
        hall1 = tc_transform(h, W1).reshape(R * N, D)
        part1 = sc_msgpass(hall1, gix2, dstp2)
        return tc_final_lin(part1, part1, h, Wself1, b1r)

    return run


def kernel(x, edge_index, edge_type, W0, Wself0, b0, W1, Wself1, b1):
    N, D = x.shape
    R = W0.shape[0]
    E = edge_index.shape[1]
    return _build(N, D, R, E)(x, edge_index, edge_type, W0, Wself0, b0,
                              W1, Wself1, b1)
